# baseline (speedup 1.0000x reference)
import jax
import jax.numpy as jnp
from jax.experimental import pallas as pl
from jax.experimental.pallas import tpu as pltpu

_BR = 12800
_KC = 2


def _mlp_block_kernel(seq_ref, w1_ref, b1_ref, w2t_ref, b2_ref, w3_ref,
                      b3_ref, out_ref, acc_ref):
    k = pl.program_id(1)
    x = seq_ref[...].astype(jnp.bfloat16)
    part = jnp.dot(x, w1_ref[...].astype(jnp.bfloat16),
                   preferred_element_type=jnp.float32)

    @pl.when(k == 0)
    def _():
        acc_ref[...] = part

    @pl.when(k > 0)
    def _():
        acc_ref[...] += part

    @pl.when(k == _KC - 1)
    def _():
        h = jnp.maximum(acc_ref[...] + b1_ref[...], 0.0).astype(jnp.bfloat16)
        h = jnp.dot(h, w2t_ref[...].astype(jnp.bfloat16).T,
                    preferred_element_type=jnp.float32)
        h = jnp.maximum(h + b2_ref[...], 0.0).astype(jnp.bfloat16)
        h = jnp.dot(h, w3_ref[...].astype(jnp.bfloat16),
                    preferred_element_type=jnp.float32)
        out_ref[...] = (h + b3_ref[...]).T


def _fused_mlp(seq, W1, b1, W2t, b2, W3, b3, *, block_rows=_BR,
               interpret=False):
    n, ft_in = seq.shape
    h1 = W1.shape[1]
    h2 = W2t.shape[0]
    nc = W3.shape[1]
    kc = ft_in // _KC
    grid = (pl.cdiv(n, block_rows), _KC)
    full = lambda shape: pl.BlockSpec(shape, lambda i, k: (0, 0))
    return pl.pallas_call(
        _mlp_block_kernel,
        grid=grid,
        in_specs=[
            pl.BlockSpec((block_rows, kc), lambda i, k: (i, k)),
            pl.BlockSpec((kc, h1), lambda i, k: (k, 0)),
            full((1, h1)),
            full((h2, h1)),
            full((1, h2)),
            full((h2, nc)),
            full((1, nc)),
        ],
        out_specs=pl.BlockSpec((nc, block_rows), lambda i, k: (0, i)),
        out_shape=jax.ShapeDtypeStruct((nc, n), seq.dtype),
        scratch_shapes=[pltpu.VMEM((block_rows, h1), jnp.float32)],
        compiler_params=pltpu.CompilerParams(
            dimension_semantics=("parallel", "arbitrary"),
            vmem_limit_bytes=100 * 1024 * 1024,
        ),
        interpret=interpret,
    )(seq, W1, b1.reshape(1, h1), W2t, b2.reshape(1, h2), W3,
      b3.reshape(1, nc))


def kernel(seq, W1, b1, W2, b2, W3, b3):
    out_t = _fused_mlp(seq, W1, b1, W2.T, b2, W3, b3)
    return out_t.T


# speedup vs baseline: 1.1374x; 1.1374x over previous
import jax
import jax.numpy as jnp
from jax.experimental import pallas as pl
from jax.experimental.pallas import tpu as pltpu

_BR = 12800


def _mlp_block_kernel(seq_ref, w1_ref, b1_ref, w2t_ref, b2_ref, w3_ref,
                      b3_ref, out_ref):
    x = seq_ref[...].astype(jnp.bfloat16)
    h = jnp.dot(x, w1_ref[...].astype(jnp.bfloat16),
                preferred_element_type=jnp.float32)
    h = jnp.maximum(h + b1_ref[...], 0.0).astype(jnp.bfloat16)
    h = jnp.dot(h, w2t_ref[...].astype(jnp.bfloat16).T,
                preferred_element_type=jnp.float32)
    h = jnp.maximum(h + b2_ref[...], 0.0).astype(jnp.bfloat16)
    h = jnp.dot(h, w3_ref[...].astype(jnp.bfloat16),
                preferred_element_type=jnp.float32)
    out_ref[...] = (h + b3_ref[...]).T


def _fused_mlp(seq, W1, b1, W2t, b2, W3, b3, *, block_rows=_BR,
               interpret=False):
    n, ft_in = seq.shape
    h1 = W1.shape[1]
    h2 = W2t.shape[0]
    nc = W3.shape[1]
    grid = (pl.cdiv(n, block_rows),)
    full = lambda shape: pl.BlockSpec(shape, lambda i: (0, 0))
    return pl.pallas_call(
        _mlp_block_kernel,
        grid=grid,
        in_specs=[
            pl.BlockSpec((block_rows, ft_in), lambda i: (i, 0)),
            full((ft_in, h1)),
            full((1, h1)),
            full((h2, h1)),
            full((1, h2)),
            full((h2, nc)),
            full((1, nc)),
        ],
        out_specs=pl.BlockSpec((nc, block_rows), lambda i: (0, i)),
        out_shape=jax.ShapeDtypeStruct((nc, n), seq.dtype),
        compiler_params=pltpu.CompilerParams(
            dimension_semantics=("parallel",),
            vmem_limit_bytes=100 * 1024 * 1024,
        ),
        interpret=interpret,
    )(seq, W1, b1.reshape(1, h1), W2t, b2.reshape(1, h2), W3,
      b3.reshape(1, nc))


def kernel(seq, W1, b1, W2, b2, W3, b3):
    out_t = _fused_mlp(seq, W1, b1, W2.T, b2, W3, b3)
    return out_t.T
